# baseline (speedup 1.0000x reference)
import jax
import jax.numpy as jnp
from jax.experimental import pallas as pl


def _node_attention_block(adj_lo_ref, adj_hi_ref, emb_ref, hv_ref, out_ref):
    emb = emb_ref[:]
    score = jnp.dot(emb, hv_ref[:],
                    preferred_element_type=jnp.float32)
    w = jnp.exp(score - jnp.max(score))
    rhs = jnp.concatenate([emb * w, w], axis=1)
    h = rhs.shape[0] // 2
    acc = (
        jnp.dot(adj_lo_ref[:], rhs[:h],
                preferred_element_type=jnp.float32)
        + jnp.dot(adj_hi_ref[:], rhs[h:],
                  preferred_element_type=jnp.float32)
    )
    out_ref[:] = acc[:, :-1] / acc[:, -1:]


@jax.jit
def kernel(emb, adj, H_v):
    n, d = emb.shape
    block_rows = 1024
    grid = (n // block_rows,)
    nh = n // 2
    return pl.pallas_call(
        _node_attention_block,
        grid=grid,
        in_specs=[
            pl.BlockSpec((block_rows, nh), lambda i: (i, 0)),
            pl.BlockSpec((block_rows, nh), lambda i: (i, 1)),
            pl.BlockSpec((n, d), lambda i: (0, 0)),
            pl.BlockSpec((d, 1), lambda i: (0, 0)),
        ],
        out_specs=pl.BlockSpec((block_rows, d), lambda i: (i, 0)),
        out_shape=jax.ShapeDtypeStruct((n, d), jnp.float32),
    )(adj, adj, emb, H_v)


# speedup vs baseline: 1.0520x; 1.0520x over previous
import jax
import jax.numpy as jnp
from jax.experimental import pallas as pl
from jax.experimental.pallas import tpu as pltpu

_N = 4096
_D = 64
_CHUNK = 512
_NCHUNKS = _N // _CHUNK


def _node_attention(adj_hbm, emb_ref, hv_ref, out_ref, buf, sem):
    emb = emb_ref[:]
    score = jnp.dot(emb, hv_ref[:],
                    preferred_element_type=jnp.float32)
    w = jnp.exp(score - jnp.max(score))
    rhs = jnp.concatenate([emb * w, w], axis=1)

    def copy_chunk(i, slot):
        return pltpu.make_async_copy(
            adj_hbm.at[pl.ds(i * _CHUNK, _CHUNK), :],
            buf.at[slot],
            sem.at[slot],
        )

    copy_chunk(0, 0).start()

    def body(i, carry):
        slot = jax.lax.rem(i, 2)

        @pl.when(i + 1 < _NCHUNKS)
        def _():
            copy_chunk(i + 1, 1 - slot).start()

        copy_chunk(i, slot).wait()
        a = buf[slot]
        acc = jnp.dot(a, rhs,
                      preferred_element_type=jnp.float32)
        out_ref[pl.ds(i * _CHUNK, _CHUNK), :] = acc[:, :-1] / acc[:, -1:]
        return carry

    jax.lax.fori_loop(0, _NCHUNKS, body, 0)


@jax.jit
def kernel(emb, adj, H_v):
    n, d = emb.shape
    return pl.pallas_call(
        _node_attention,
        in_specs=[
            pl.BlockSpec(memory_space=pltpu.MemorySpace.HBM),
            pl.BlockSpec(memory_space=pltpu.MemorySpace.VMEM),
            pl.BlockSpec(memory_space=pltpu.MemorySpace.VMEM),
        ],
        out_specs=pl.BlockSpec(memory_space=pltpu.MemorySpace.VMEM),
        out_shape=jax.ShapeDtypeStruct((n, d), jnp.float32),
        scratch_shapes=[
            pltpu.VMEM((2, _CHUNK, _N), jnp.float32),
            pltpu.SemaphoreType.DMA((2,)),
        ],
    )(adj, emb, H_v)
